# baseline (speedup 1.0000x reference)
import functools

import jax
import jax.numpy as jnp
from jax.experimental import pallas as pl
from jax.experimental.pallas import tpu as pltpu

_NSEG = 512
_SEGCHUNK = 128
_NCHUNK = _NSEG // _SEGCHUNK


def _body(batch_ref, x_ref, w1_ref, b1_ref, w2_ref, b2_ref, out_ref, den_ref,
          *, nblocks, bn):
    blk = pl.program_id(0)

    @pl.when(blk == 0)
    def _init():
        out_ref[...] = jnp.zeros_like(out_ref)
        den_ref[...] = jnp.zeros_like(den_ref)

    xb16 = x_ref[...].astype(jnp.bfloat16)
    h = jnp.tanh(
        jax.lax.dot_general(xb16, w1_ref[...], (((1,), (0,)), ((), ())),
                            preferred_element_type=jnp.float32)
        + b1_ref[...])
    lm = jax.lax.dot_general(h.astype(jnp.bfloat16), w2_ref[...],
                             (((1,), (0,)), ((), ())),
                             preferred_element_type=jnp.float32)
    em16 = jnp.exp(lm + b2_ref[0, 0]).astype(jnp.bfloat16)

    b = batch_ref[...]
    bmin = batch_ref[0, 0]
    bmax = batch_ref[bn - 1, 0]
    ones_row = jnp.ones((1, bn), jnp.bfloat16)

    for c in range(_NCHUNK):
        @pl.when((bmin < (c + 1) * _SEGCHUNK) & (bmax >= c * _SEGCHUNK))
        def _chunk(c=c):
            seg_ids = (jax.lax.broadcasted_iota(jnp.int32, (bn, _SEGCHUNK), 1)
                       + c * _SEGCHUNK)
            ow = jnp.where(b == seg_ids, em16, jnp.bfloat16(0))
            num = jax.lax.dot_general(ow, xb16, (((0,), (0,)), ((), ())),
                                      preferred_element_type=jnp.float32)
            dsum = jax.lax.dot_general(ones_row, ow, (((1,), (0,)), ((), ())),
                                       preferred_element_type=jnp.float32)
            sl = pl.ds(c * _SEGCHUNK, _SEGCHUNK)
            out_ref[sl, :] = out_ref[sl, :] + num
            den_ref[c:c + 1, :] = den_ref[c:c + 1, :] + dsum

    @pl.when(blk == nblocks - 1)
    def _finish():
        for c in range(_NCHUNK):
            sl = pl.ds(c * _SEGCHUNK, _SEGCHUNK)
            dcol = jnp.transpose(den_ref[c:c + 1, :])
            out_ref[sl, :] = out_ref[sl, :] / (dcol + 1e-16)


def kernel(x, batch, W1, b1, W2, b2):
    n, d = x.shape
    bn = 4000
    nblocks = pl.cdiv(n, bn)

    batch2d = batch.reshape(n, 1)
    b1r = b1.reshape(1, d)
    w2rep = jnp.broadcast_to(W2, (d, d)).astype(jnp.bfloat16)
    b2r = b2.reshape(1, 1)
    w1_16 = W1.astype(jnp.bfloat16)

    out = pl.pallas_call(
        functools.partial(_body, nblocks=nblocks, bn=bn),
        grid=(nblocks,),
        in_specs=[
            pl.BlockSpec((bn, 1), lambda i: (i, 0)),
            pl.BlockSpec((bn, d), lambda i: (i, 0)),
            pl.BlockSpec((d, d), lambda i: (0, 0)),
            pl.BlockSpec((1, d), lambda i: (0, 0)),
            pl.BlockSpec((d, d), lambda i: (0, 0)),
            pl.BlockSpec((1, 1), lambda i: (0, 0)),
        ],
        out_specs=pl.BlockSpec((_NSEG, d), lambda i: (0, 0)),
        out_shape=jax.ShapeDtypeStruct((_NSEG, d), jnp.float32),
        scratch_shapes=[pltpu.VMEM((8, _SEGCHUNK), jnp.float32)],
        compiler_params=pltpu.CompilerParams(
            dimension_semantics=("arbitrary",),
        ),
    )(batch2d, x, w1_16, b1r, w2rep, b2r)
    return out


# speedup vs baseline: 1.0010x; 1.0010x over previous
import functools

import jax
import jax.numpy as jnp
from jax.experimental import pallas as pl
from jax.experimental.pallas import tpu as pltpu

_NSEG = 512
_SEGCHUNK = 128
_NCHUNK = _NSEG // _SEGCHUNK
_ACC_ROWS = _NSEG + _SEGCHUNK


def _body(batch_ref, x_ref, w1_ref, b1_ref, w2_ref, b2_ref, out_ref,
          acc_ref, den_ref, *, nblocks, bn):
    blk = pl.program_id(0)

    @pl.when(blk == 0)
    def _init():
        acc_ref[...] = jnp.zeros_like(acc_ref)
        den_ref[...] = jnp.zeros_like(den_ref)

    xb16 = x_ref[...].astype(jnp.bfloat16)
    h = jnp.tanh(
        jax.lax.dot_general(xb16, w1_ref[...], (((1,), (0,)), ((), ())),
                            preferred_element_type=jnp.float32)
        + b1_ref[...])
    lm = jax.lax.dot_general(h.astype(jnp.bfloat16), w2_ref[...],
                             (((1,), (0,)), ((), ())),
                             preferred_element_type=jnp.float32)
    em16 = jnp.exp(lm + b2_ref[0, 0]).astype(jnp.bfloat16)

    b = batch_ref[...]
    bmin = batch_ref[0, 0]
    bmax = batch_ref[bn - 1, 0]
    base = (bmin // 8) * 8
    ones_row = jnp.ones((1, bn), jnp.bfloat16)
    lane = jax.lax.broadcasted_iota(jnp.int32, (bn, _SEGCHUNK), 1)

    @pl.when(bmax - base < _SEGCHUNK)
    def _fast():
        ow = jnp.where(b - base == lane, em16, jnp.bfloat16(0))
        num = jax.lax.dot_general(ow, xb16, (((0,), (0,)), ((), ())),
                                  preferred_element_type=jnp.float32)
        dsum = jax.lax.dot_general(ones_row, ow, (((1,), (0,)), ((), ())),
                                   preferred_element_type=jnp.float32)
        dcol = jnp.transpose(dsum)
        sl = pl.ds(base, _SEGCHUNK)
        acc_ref[sl, :] = acc_ref[sl, :] + num
        den_ref[sl, :] = den_ref[sl, :] + dcol

    @pl.when(bmax - base >= _SEGCHUNK)
    def _slow():
        for c in range(_NCHUNK):
            @pl.when((bmin < (c + 1) * _SEGCHUNK) & (bmax >= c * _SEGCHUNK))
            def _chunk(c=c):
                ow = jnp.where(b - c * _SEGCHUNK == lane, em16,
                               jnp.bfloat16(0))
                num = jax.lax.dot_general(ow, xb16, (((0,), (0,)), ((), ())),
                                          preferred_element_type=jnp.float32)
                dsum = jax.lax.dot_general(ones_row, ow,
                                           (((1,), (0,)), ((), ())),
                                           preferred_element_type=jnp.float32)
                dcol = jnp.transpose(dsum)
                sl = pl.ds(c * _SEGCHUNK, _SEGCHUNK)
                acc_ref[sl, :] = acc_ref[sl, :] + num
                den_ref[sl, :] = den_ref[sl, :] + dcol

    @pl.when(blk == nblocks - 1)
    def _finish():
        out_ref[...] = acc_ref[0:_NSEG, :] / (den_ref[0:_NSEG, :] + 1e-16)


def kernel(x, batch, W1, b1, W2, b2):
    n, d = x.shape
    bn = 4000
    nblocks = pl.cdiv(n, bn)

    batch2d = batch.reshape(n, 1)
    b1r = b1.reshape(1, d)
    w2rep = jnp.broadcast_to(W2, (d, d)).astype(jnp.bfloat16)
    b2r = b2.reshape(1, 1)
    w1_16 = W1.astype(jnp.bfloat16)

    out = pl.pallas_call(
        functools.partial(_body, nblocks=nblocks, bn=bn),
        grid=(nblocks,),
        in_specs=[
            pl.BlockSpec((bn, 1), lambda i: (i, 0)),
            pl.BlockSpec((bn, d), lambda i: (i, 0)),
            pl.BlockSpec((d, d), lambda i: (0, 0)),
            pl.BlockSpec((1, d), lambda i: (0, 0)),
            pl.BlockSpec((d, d), lambda i: (0, 0)),
            pl.BlockSpec((1, 1), lambda i: (0, 0)),
        ],
        out_specs=pl.BlockSpec((_NSEG, d), lambda i: (0, 0)),
        out_shape=jax.ShapeDtypeStruct((_NSEG, d), jnp.float32),
        scratch_shapes=[
            pltpu.VMEM((_ACC_ROWS, d), jnp.float32),
            pltpu.VMEM((_ACC_ROWS, 1), jnp.float32),
        ],
        compiler_params=pltpu.CompilerParams(
            dimension_semantics=("arbitrary",),
        ),
    )(batch2d, x, w1_16, b1r, w2rep, b2r)
    return out


# speedup vs baseline: 2.1048x; 2.1026x over previous
import functools

import jax
import jax.numpy as jnp
from jax.experimental import pallas as pl
from jax.experimental.pallas import tpu as pltpu

_NSEG = 512
_SEGCHUNK = 128
_NCHUNK = _NSEG // _SEGCHUNK
_ACC_ROWS = _NSEG + _SEGCHUNK


def _body(batch_ref, x_ref, w1_ref, b1_ref, w2_ref, b2_ref, out_ref,
          acc_ref, den_ref, *, nblocks, bn):
    blk = pl.program_id(0)

    @pl.when(blk == 0)
    def _init():
        acc_ref[...] = jnp.zeros_like(acc_ref)
        den_ref[...] = jnp.zeros_like(den_ref)

    xb16 = x_ref[...].astype(jnp.bfloat16)
    h = jnp.tanh(
        jax.lax.dot_general(xb16, w1_ref[...], (((1,), (0,)), ((), ())),
                            preferred_element_type=jnp.float32)
        + b1_ref[...])
    lrow = jax.lax.dot_general(w2_ref[...], h.astype(jnp.bfloat16),
                               (((0,), (1,)), ((), ())),
                               preferred_element_type=jnp.float32)
    erow = jnp.exp(lrow + b2_ref[0, 0]).astype(jnp.bfloat16)

    brow = batch_ref[0]
    bmin = batch_ref[0, 0, 0]
    bmax = batch_ref[0, 0, bn - 1]
    base = (bmin // 8) * 8
    ones8 = jnp.ones((bn, 8), jnp.bfloat16)
    subl = jax.lax.broadcasted_iota(jnp.int16, (_SEGCHUNK, bn), 0)

    def _scatter(anchor, sl):
        rel = (brow - anchor).astype(jnp.int16)
        owt = jnp.where(rel == subl, erow, jnp.bfloat16(0))
        num = jax.lax.dot_general(owt, xb16, (((1,), (0,)), ((), ())),
                                  preferred_element_type=jnp.float32)
        dcol = jax.lax.dot_general(owt, ones8, (((1,), (0,)), ((), ())),
                                   preferred_element_type=jnp.float32)
        acc_ref[sl, :] = acc_ref[sl, :] + num
        den_ref[sl, :] = den_ref[sl, :] + dcol[:, 0:1]

    @pl.when(bmax - base < _SEGCHUNK)
    def _fast():
        _scatter(base, pl.ds(base, _SEGCHUNK))

    @pl.when(bmax - base >= _SEGCHUNK)
    def _slow():
        for c in range(_NCHUNK):
            @pl.when((bmin < (c + 1) * _SEGCHUNK) & (bmax >= c * _SEGCHUNK))
            def _chunk(c=c):
                _scatter(c * _SEGCHUNK, pl.ds(c * _SEGCHUNK, _SEGCHUNK))

    @pl.when(blk == nblocks - 1)
    def _finish():
        out_ref[...] = acc_ref[0:_NSEG, :] / (den_ref[0:_NSEG, :] + 1e-16)


def kernel(x, batch, W1, b1, W2, b2):
    n, d = x.shape
    bn = 4000
    nblocks = pl.cdiv(n, bn)

    batch3d = batch.reshape(nblocks, 1, bn)
    b1r = b1.reshape(1, d)
    w2col = W2.astype(jnp.bfloat16)
    b2r = b2.reshape(1, 1)
    w1_16 = W1.astype(jnp.bfloat16)

    out = pl.pallas_call(
        functools.partial(_body, nblocks=nblocks, bn=bn),
        grid=(nblocks,),
        in_specs=[
            pl.BlockSpec((1, 1, bn), lambda i: (i, 0, 0)),
            pl.BlockSpec((bn, d), lambda i: (i, 0)),
            pl.BlockSpec((d, d), lambda i: (0, 0)),
            pl.BlockSpec((1, d), lambda i: (0, 0)),
            pl.BlockSpec((d, 1), lambda i: (0, 0)),
            pl.BlockSpec((1, 1), lambda i: (0, 0)),
        ],
        out_specs=pl.BlockSpec((_NSEG, d), lambda i: (0, 0)),
        out_shape=jax.ShapeDtypeStruct((_NSEG, d), jnp.float32),
        scratch_shapes=[
            pltpu.VMEM((_ACC_ROWS, d), jnp.float32),
            pltpu.VMEM((_ACC_ROWS, 1), jnp.float32),
        ],
        compiler_params=pltpu.CompilerParams(
            dimension_semantics=("arbitrary",),
        ),
    )(batch3d, x, w1_16, b1r, w2col, b2r)
    return out
